# speedup vs baseline: 1.1559x; 1.1559x over previous
import functools

import jax
import jax.numpy as jnp
from jax import lax
from jax.experimental import pallas as pl
from jax.experimental.pallas import tpu as pltpu
from jax.experimental.pallas import tpu_sc as plsc

_INFO = plsc.get_sparse_core_info()
_NC = _INFO.num_cores
_NS = _INFO.num_subcores
_NW = _NC * _NS


def _make_sc_gather(V, D, B):
    assert B % _NW == 0
    b_per_w = B // _NW
    mesh = plsc.VectorSubcoreMesh(core_axis_name="c", subcore_axis_name="s")

    @functools.partial(
        pl.kernel,
        mesh=mesh,
        out_type=(
            jax.ShapeDtypeStruct((B, D), jnp.float32),
            jax.ShapeDtypeStruct((B, D), jnp.float32),
        ),
        scratch_types=[
            pltpu.VMEM((b_per_w,), jnp.int32),
            pltpu.VMEM((b_per_w, D), jnp.float32),
            pltpu.SemaphoreType.DMA,
        ],
    )
    def sc_gather(table_hbm, idx1_hbm, idx2_hbm, e1_hbm, e2_hbm,
                  idx_v, rows_v, sem):
        wid = lax.axis_index("s") * _NC + lax.axis_index("c")
        base = wid * b_per_w
        pltpu.sync_copy(idx1_hbm.at[pl.ds(base, b_per_w)], idx_v)
        pltpu.async_copy(table_hbm.at[idx_v], rows_v, sem).wait()
        pltpu.sync_copy(rows_v, e1_hbm.at[pl.ds(base, b_per_w)])
        pltpu.sync_copy(idx2_hbm.at[pl.ds(base, b_per_w)], idx_v)
        pltpu.async_copy(table_hbm.at[idx_v], rows_v, sem).wait()
        pltpu.sync_copy(rows_v, e2_hbm.at[pl.ds(base, b_per_w)])

    return sc_gather


def _concat_body(alpha_ref, e1_ref, e2_ref, feat_ref, out_ref):
    a = alpha_ref[0, 0]
    emb = a * e1_ref[...] + (1.0 - a) * e2_ref[...]
    out_ref[:, 0:1, :] = emb[:, None, :]
    out_ref[:, 1:, :] = feat_ref[...]


def kernel(feature, index_value_1, index_value_2, embedding_table, alpha):
    B, T, D = feature.shape
    V = embedding_table.shape[0]
    idx1 = index_value_1.astype(jnp.int32)
    idx2 = index_value_2.astype(jnp.int32)

    e1, e2 = _make_sc_gather(V, D, B)(embedding_table, idx1, idx2)

    BB = 32
    grid = (B // BB,)
    out = pl.pallas_call(
        _concat_body,
        grid=grid,
        in_specs=[
            pl.BlockSpec(memory_space=pltpu.SMEM),
            pl.BlockSpec((BB, D), lambda i: (i, 0)),
            pl.BlockSpec((BB, D), lambda i: (i, 0)),
            pl.BlockSpec((BB, T, D), lambda i: (i, 0, 0)),
        ],
        out_specs=pl.BlockSpec((BB, T + 1, D), lambda i: (i, 0, 0)),
        out_shape=jax.ShapeDtypeStruct((B, T + 1, D), jnp.float32),
        compiler_params=pltpu.CompilerParams(
            dimension_semantics=("parallel",),
        ),
    )(jnp.reshape(alpha.astype(jnp.float32), (1, 1)), e1, e2, feature)
    return out


# speedup vs baseline: 1.1726x; 1.0144x over previous
import functools

import jax
import jax.numpy as jnp
from jax import lax
from jax.experimental import pallas as pl
from jax.experimental.pallas import tpu as pltpu
from jax.experimental.pallas import tpu_sc as plsc

_INFO = plsc.get_sparse_core_info()
_NC = _INFO.num_cores
_NS = _INFO.num_subcores
_NW = _NC * _NS


def _make_sc_gather(V, D, B):
    assert B % _NW == 0
    b_per_w = B // _NW
    mesh = plsc.VectorSubcoreMesh(core_axis_name="c", subcore_axis_name="s")

    @functools.partial(
        pl.kernel,
        mesh=mesh,
        out_type=(
            jax.ShapeDtypeStruct((B, D), jnp.float32),
            jax.ShapeDtypeStruct((B, D), jnp.float32),
        ),
        scratch_types=[
            pltpu.VMEM((b_per_w,), jnp.int32),
            pltpu.VMEM((b_per_w, D), jnp.float32),
            pltpu.SemaphoreType.DMA,
        ],
    )
    def sc_gather(table_hbm, idx1_hbm, idx2_hbm, e1_hbm, e2_hbm,
                  idx_v, rows_v, sem):
        wid = lax.axis_index("s") * _NC + lax.axis_index("c")
        base = wid * b_per_w
        pltpu.sync_copy(idx1_hbm.at[pl.ds(base, b_per_w)], idx_v)
        pltpu.async_copy(table_hbm.at[idx_v], rows_v, sem).wait()
        pltpu.sync_copy(rows_v, e1_hbm.at[pl.ds(base, b_per_w)])
        pltpu.sync_copy(idx2_hbm.at[pl.ds(base, b_per_w)], idx_v)
        pltpu.async_copy(table_hbm.at[idx_v], rows_v, sem).wait()
        pltpu.sync_copy(rows_v, e2_hbm.at[pl.ds(base, b_per_w)])

    return sc_gather


def _concat_body(alpha_ref, e1_ref, e2_ref, feat_ref, out_ref):
    a = alpha_ref[0, 0]
    emb = a * e1_ref[...] + (1.0 - a) * e2_ref[...]
    out_ref[:, 0:1, :] = emb[:, None, :]
    out_ref[:, 1:, :] = feat_ref[...]


def kernel(feature, index_value_1, index_value_2, embedding_table, alpha):
    B, T, D = feature.shape
    V = embedding_table.shape[0]
    idx1 = index_value_1.astype(jnp.int32)
    idx2 = index_value_2.astype(jnp.int32)

    e1, e2 = _make_sc_gather(V, D, B)(embedding_table, idx1, idx2)

    BB = 64
    grid = (B // BB,)
    out = pl.pallas_call(
        _concat_body,
        grid=grid,
        in_specs=[
            pl.BlockSpec(memory_space=pltpu.SMEM),
            pl.BlockSpec((BB, D), lambda i: (i, 0)),
            pl.BlockSpec((BB, D), lambda i: (i, 0)),
            pl.BlockSpec((BB, T, D), lambda i: (i, 0, 0)),
        ],
        out_specs=pl.BlockSpec((BB, T + 1, D), lambda i: (i, 0, 0)),
        out_shape=jax.ShapeDtypeStruct((B, T + 1, D), jnp.float32),
        compiler_params=pltpu.CompilerParams(
            dimension_semantics=("parallel",),
        ),
    )(jnp.reshape(alpha.astype(jnp.float32), (1, 1)), e1, e2, feature)
    return out
